# Initial kernel scaffold; baseline (speedup 1.0000x reference)
import functools
import math

import jax
import jax.numpy as jnp
import numpy as np
from jax.experimental import pallas as pl

_HID = 128
_NH = 4
_DH = 32
_K = 30
_NPOS = 16
_BND = 32


def _dot_t(x, w):
    return jax.lax.dot_general(x, w, (((1,), (1,)), ((), ())),
                               preferred_element_type=jnp.float32)


def _dot_n(x, w):
    return jax.lax.dot_general(x, w, (((1,), (0,)), ((), ())),
                               preferred_element_type=jnp.float32)


def _ln(x, g, b):
    mu = jnp.mean(x, axis=1, keepdims=True)
    xc = x - mu
    var = jnp.mean(xc * xc, axis=1, keepdims=True)
    return xc * jax.lax.rsqrt(var + 1e-5) * g + b


def _body(s_ref, m_ref, we_ref, be_ref, ws_ref, wout_ref, bout_ref,
          wq_ref, wke_ref, wks_ref, wkv_ref, wve_ref, wvs_ref, wvv_ref,
          wo_ref, w1_ref, b1_ref, w2_ref, b2_ref,
          ln1g_ref, ln1b_ref, ln2g_ref, ln2b_ref,
          fint_ref, fbnd_ref, mbnd_ref, bsel_ref, out_ref, *, num_layers):
    N = s_ref.shape[0]
    scale = 1.0 / math.sqrt(_DH)

    s = s_ref[...]
    vocab = ws_ref.shape[0]
    onehot = (jax.lax.broadcasted_iota(jnp.int32, (N, vocab), 1) == s
              ).astype(jnp.float32)
    hS = _dot_n(onehot, ws_ref[...])

    hEi = _dot_t(fint_ref[...], we_ref[...]) + be_ref[...]
    hEb = _dot_t(fbnd_ref[...], we_ref[...]) + be_ref[...]

    seq_mask = m_ref[...]
    mfull = jnp.concatenate(
        [mbnd_ref[...], jnp.ones((N - _BND, _K), jnp.float32)], axis=0)

    bsel = bsel_ref[...]
    hV = jnp.zeros((N, _HID), jnp.float32)

    for l in range(num_layers):
        Q = _dot_t(hV, wq_ref[l]) * scale
        KP = _dot_t(hS, wks_ref[l]) + _dot_t(hV, wkv_ref[l])
        VP = _dot_t(hS, wvs_ref[l]) + _dot_t(hV, wvv_ref[l])
        EK = _dot_t(hEi, wke_ref[l])
        EKb = _dot_t(hEb, wke_ref[l])
        EV = _dot_t(hEi, wve_ref[l])
        EVb = _dot_t(hEb, wve_ref[l])
        KR32 = _dot_n(bsel, KP[0:_BND, :])
        VR32 = _dot_n(bsel, VP[0:_BND, :])

        logits_h = []
        for h in range(_NH):
            sl = slice(h * _DH, (h + 1) * _DH)
            Qh = Q[:, sl]
            lint = _dot_t(Qh[_BND:, :], EK[:, sl])
            pb = (Qh[0:_BND, :].reshape(_BND, 1, _DH)
                  * EKb[:, sl].reshape(_BND, _K, _DH))
            lbnd = jnp.sum(pb, axis=2)
            logits_h.append(jnp.concatenate([lbnd, lint], axis=0))

        lk_cols = [[] for _ in range(_NH)]
        for k in range(_K):
            KRk = jnp.concatenate(
                [KR32[k * _BND:(k + 1) * _BND, :],
                 KP[(_BND - 1 - k):(N - 1 - k), :]], axis=0)
            prod = Q * KRk
            for h in range(_NH):
                lk_cols[h].append(jnp.sum(prod[:, h * _DH:(h + 1) * _DH],
                                          axis=1, keepdims=True))

        A_h = []
        for h in range(_NH):
            lg = logits_h[h] + jnp.concatenate(lk_cols[h], axis=1)
            lg = jnp.where(mfull > 0.0, lg, -1e9)
            mx = jnp.max(lg, axis=1, keepdims=True)
            e = jnp.exp(lg - mx)
            A_h.append(e / jnp.sum(e, axis=1, keepdims=True) * mfull)

        acc = []
        for h in range(_NH):
            sl = slice(h * _DH, (h + 1) * _DH)
            aint = _dot_n(A_h[h][_BND:, :], EV[:, sl])
            pbv = (A_h[h][0:_BND, :].reshape(_BND, _K, 1)
                   * EVb[:, sl].reshape(_BND, _K, _DH))
            abnd = jnp.sum(pbv, axis=1)
            acc.append(jnp.concatenate([abnd, aint], axis=0))
        for k in range(_K):
            VRk = jnp.concatenate(
                [VR32[k * _BND:(k + 1) * _BND, :],
                 VP[(_BND - 1 - k):(N - 1 - k), :]], axis=0)
            for h in range(_NH):
                acc[h] = acc[h] + A_h[h][:, k:k + 1] * VRk[:, h * _DH:(h + 1) * _DH]
        h_att = jnp.concatenate(acc, axis=1)

        hV = _ln(hV + _dot_t(h_att, wo_ref[l]), ln1g_ref[l], ln1b_ref[l])
        ffn = _dot_t(jax.nn.relu(_dot_t(hV, w1_ref[l]) + b1_ref[l]),
                     w2_ref[l]) + b2_ref[l]
        hV = _ln(hV + ffn, ln2g_ref[l], ln2b_ref[l])
        hV = hV * seq_mask

    logits_out = _dot_t(hV, wout_ref[...]) + bout_ref[...]
    mx = jnp.max(logits_out, axis=1, keepdims=True)
    lse = mx + jnp.log(jnp.sum(jnp.exp(logits_out - mx), axis=1, keepdims=True))
    out_ref[...] = (logits_out - lse).reshape(1, N, logits_out.shape[1])


def _band_constants():
    freq = np.exp(np.arange(0, _NPOS, 2, dtype=np.float64)
                  * (-np.log(10000.0) / _NPOS))

    def feat(d):
        ang = d[..., None] * freq
        return np.concatenate([np.cos(ang), np.sin(ang)],
                              axis=-1).astype(np.float32)

    d_int = -(np.arange(_K, dtype=np.float64) + 1.0)
    feat_int = feat(d_int)

    nn = np.arange(_BND)[:, None]
    kk = np.arange(_K)[None, :]
    jsel = np.abs(nn - kk - 1)
    dbnd = (jsel - nn).astype(np.float64)
    feat_bnd = feat(dbnd).reshape(_BND * _K, _NPOS)
    mask_bnd = (dbnd < 0).astype(np.float32)

    bsel = np.zeros((_K, _BND, _BND), np.float32)
    for k in range(_K):
        for n in range(_BND):
            bsel[k, n, jsel[n, k]] = 1.0
    bsel = bsel.reshape(_K * _BND, _BND)
    return feat_int, feat_bnd, mask_bnd, bsel


@jax.jit
def kernel(S, L, mask, params):
    del L
    p = params
    B, N = S.shape
    num_layers = p['W_Q'].shape[0]

    feat_int, feat_bnd, mask_bnd, bsel = _band_constants()

    wke = p['W_K'][:, :, 0:_HID]
    wks = p['W_K'][:, :, _HID:2 * _HID]
    wkv = p['W_K'][:, :, 2 * _HID:3 * _HID]
    wve = p['W_V'][:, :, 0:_HID]
    wvs = p['W_V'][:, :, _HID:2 * _HID]
    wvv = p['W_V'][:, :, 2 * _HID:3 * _HID]

    full = lambda a: pl.BlockSpec(a.shape, lambda b: (0,) * a.ndim)
    args = [
        S.T, mask.T,
        p['W_e'], p['b_e'].reshape(1, _HID),
        p['W_s'], p['W_out'], p['b_out'].reshape(1, -1),
        p['W_Q'], wke, wks, wkv, wve, wvs, wvv, p['W_O'],
        p['W_1'], p['b_1'].reshape(num_layers, 1, -1),
        p['W_2'], p['b_2'].reshape(num_layers, 1, -1),
        p['ln1_g'].reshape(num_layers, 1, -1),
        p['ln1_b'].reshape(num_layers, 1, -1),
        p['ln2_g'].reshape(num_layers, 1, -1),
        p['ln2_b'].reshape(num_layers, 1, -1),
        jnp.asarray(feat_int), jnp.asarray(feat_bnd),
        jnp.asarray(mask_bnd), jnp.asarray(bsel),
    ]
    in_specs = [
        pl.BlockSpec((N, 1), lambda b: (0, b)),
        pl.BlockSpec((N, 1), lambda b: (0, b)),
    ] + [full(a) for a in args[2:]]

    out = pl.pallas_call(
        functools.partial(_body, num_layers=num_layers),
        grid=(B,),
        in_specs=in_specs,
        out_specs=pl.BlockSpec((1, N, 20), lambda b: (b, 0, 0)),
        out_shape=jax.ShapeDtypeStruct((B, N, 20), jnp.float32),
    )(*args)
    return out


# speedup vs baseline: 5.4920x; 5.4920x over previous
import functools
import math

import jax
import jax.numpy as jnp
import numpy as np
from jax.experimental import pallas as pl

_HID = 128
_NH = 4
_DH = 32
_K = 30
_NPOS = 16
_BND = 32


def _dot_t(x, w):
    return jax.lax.dot_general(x, w, (((1,), (1,)), ((), ())),
                               preferred_element_type=jnp.float32)


def _dot_n(x, w):
    return jax.lax.dot_general(x, w, (((1,), (0,)), ((), ())),
                               preferred_element_type=jnp.float32)


def _ln(x, g, b):
    mu = jnp.mean(x, axis=1, keepdims=True)
    xc = x - mu
    var = jnp.mean(xc * xc, axis=1, keepdims=True)
    return xc * jax.lax.rsqrt(var + 1e-5) * g + b


def _body(s_ref, m_ref, we_ref, be_ref, ws_ref, wout_ref, bout_ref,
          wq_ref, wke_ref, wks_ref, wkv_ref, wve_ref, wvs_ref, wvv_ref,
          wo_ref, w1_ref, b1_ref, w2_ref, b2_ref,
          ln1g_ref, ln1b_ref, ln2g_ref, ln2b_ref,
          fint_ref, fbnd_ref, mbnd_ref, bsel_ref, out_ref, *, num_layers):
    N = s_ref.shape[1]
    scale = 1.0 / math.sqrt(_DH)

    s = s_ref[0]
    vocab = ws_ref.shape[0]
    onehot = (jax.lax.broadcasted_iota(jnp.int32, (N, vocab), 1) == s
              ).astype(jnp.float32)
    hS = _dot_n(onehot, ws_ref[...])

    hEi = _dot_t(fint_ref[...], we_ref[...]) + be_ref[...]
    hEb = _dot_t(fbnd_ref[...], we_ref[...]) + be_ref[...]

    seq_mask = m_ref[0]
    mfull = jnp.concatenate(
        [mbnd_ref[...], jnp.ones((N - _BND, _K), jnp.float32)], axis=0)

    bsel = bsel_ref[...]
    hV = jnp.zeros((N, _HID), jnp.float32)

    for l in range(num_layers):
        Q = _dot_t(hV, wq_ref[l]) * scale
        KP = _dot_t(hS, wks_ref[l]) + _dot_t(hV, wkv_ref[l])
        VP = _dot_t(hS, wvs_ref[l]) + _dot_t(hV, wvv_ref[l])
        EK = _dot_t(hEi, wke_ref[l])
        EKb = _dot_t(hEb, wke_ref[l])
        EV = _dot_t(hEi, wve_ref[l])
        EVb = _dot_t(hEb, wve_ref[l])
        KR32 = _dot_n(bsel, KP[0:_BND, :])
        VR32 = _dot_n(bsel, VP[0:_BND, :])

        def band_row(k, base, e_int, e_bnd, r32):
            top = (r32[k * _BND:(k + 1) * _BND, :]
                   + e_bnd[k * _BND:(k + 1) * _BND, :])
            rest = base[(_BND - 1 - k):(N - 1 - k), :] + e_int[k:k + 1, :]
            return jnp.concatenate([top, rest], axis=0)

        def logit4(k):
            kt = band_row(k, KP, EK, EKb, KR32)
            l4 = jnp.sum((Q * kt).reshape(N, _NH, _DH), axis=2)
            mcol = mfull[:, k:k + 1]
            return jnp.where(mcol > 0.0, l4, -1e9), mcol

        m_run = jnp.full((N, _NH), -3.0e38, jnp.float32)
        s_run = jnp.zeros((N, _NH), jnp.float32)
        for k in range(_K):
            l4, _ = logit4(k)
            m_new = jnp.maximum(m_run, l4)
            s_run = s_run * jnp.exp(m_run - m_new) + jnp.exp(l4 - m_new)
            m_run = m_new
        inv_s = 1.0 / s_run

        acc = jnp.zeros((N, _HID), jnp.float32)
        for k in range(_K):
            l4, mcol = logit4(k)
            a4 = jnp.exp(l4 - m_run) * inv_s * mcol
            a128 = jnp.broadcast_to(a4.reshape(N, _NH, 1),
                                    (N, _NH, _DH)).reshape(N, _HID)
            acc = acc + a128 * band_row(k, VP, EV, EVb, VR32)
        h_att = acc

        hV = _ln(hV + _dot_t(h_att, wo_ref[l]), ln1g_ref[l], ln1b_ref[l])
        ffn = _dot_t(jax.nn.relu(_dot_t(hV, w1_ref[l]) + b1_ref[l]),
                     w2_ref[l]) + b2_ref[l]
        hV = _ln(hV + ffn, ln2g_ref[l], ln2b_ref[l])
        hV = hV * seq_mask

    logits_out = _dot_t(hV, wout_ref[...]) + bout_ref[...]
    mx = jnp.max(logits_out, axis=1, keepdims=True)
    lse = mx + jnp.log(jnp.sum(jnp.exp(logits_out - mx), axis=1, keepdims=True))
    out_ref[...] = (logits_out - lse).reshape(1, N, logits_out.shape[1])


def _band_constants():
    freq = np.exp(np.arange(0, _NPOS, 2, dtype=np.float64)
                  * (-np.log(10000.0) / _NPOS))

    def feat(d):
        ang = d[..., None] * freq
        return np.concatenate([np.cos(ang), np.sin(ang)],
                              axis=-1).astype(np.float32)

    d_int = -(np.arange(_K, dtype=np.float64) + 1.0)
    feat_int = feat(d_int)

    nn = np.arange(_BND)[:, None]
    kk = np.arange(_K)[None, :]
    jsel = np.abs(nn - kk - 1)
    dbnd = (jsel - nn).astype(np.float64)
    feat_bnd = feat(dbnd.T).reshape(_K * _BND, _NPOS)
    mask_bnd = (dbnd < 0).astype(np.float32)

    bsel = np.zeros((_K, _BND, _BND), np.float32)
    for k in range(_K):
        for n in range(_BND):
            bsel[k, n, jsel[n, k]] = 1.0
    bsel = bsel.reshape(_K * _BND, _BND)
    return feat_int, feat_bnd, mask_bnd, bsel


@jax.jit
def kernel(S, L, mask, params):
    del L
    p = params
    B, N = S.shape
    num_layers = p['W_Q'].shape[0]

    feat_int, feat_bnd, mask_bnd, bsel = _band_constants()

    wke = p['W_K'][:, :, 0:_HID]
    wks = p['W_K'][:, :, _HID:2 * _HID]
    wkv = p['W_K'][:, :, 2 * _HID:3 * _HID]
    wve = p['W_V'][:, :, 0:_HID]
    wvs = p['W_V'][:, :, _HID:2 * _HID]
    wvv = p['W_V'][:, :, 2 * _HID:3 * _HID]

    full = lambda a: pl.BlockSpec(a.shape, lambda b: (0,) * a.ndim)
    args = [
        S.reshape(B, N, 1), mask.reshape(B, N, 1),
        p['W_e'], p['b_e'].reshape(1, _HID),
        p['W_s'], p['W_out'], p['b_out'].reshape(1, -1),
        p['W_Q'], wke, wks, wkv, wve, wvs, wvv, p['W_O'],
        p['W_1'], p['b_1'].reshape(num_layers, 1, -1),
        p['W_2'], p['b_2'].reshape(num_layers, 1, -1),
        p['ln1_g'].reshape(num_layers, 1, -1),
        p['ln1_b'].reshape(num_layers, 1, -1),
        p['ln2_g'].reshape(num_layers, 1, -1),
        p['ln2_b'].reshape(num_layers, 1, -1),
        jnp.asarray(feat_int), jnp.asarray(feat_bnd),
        jnp.asarray(mask_bnd), jnp.asarray(bsel),
    ]
    in_specs = [
        pl.BlockSpec((1, N, 1), lambda b: (b, 0, 0)),
        pl.BlockSpec((1, N, 1), lambda b: (b, 0, 0)),
    ] + [full(a) for a in args[2:]]

    out = pl.pallas_call(
        functools.partial(_body, num_layers=num_layers),
        grid=(B,),
        in_specs=in_specs,
        out_specs=pl.BlockSpec((1, N, 20), lambda b: (b, 0, 0)),
        out_shape=jax.ShapeDtypeStruct((B, N, 20), jnp.float32),
    )(*args)
    return out


# speedup vs baseline: 5.9994x; 1.0924x over previous
import functools
import math

import jax
import jax.numpy as jnp
import numpy as np
from jax.experimental import pallas as pl
from jax.experimental.pallas import tpu as pltpu

_HID = 128
_NH = 4
_DH = 32
_K = 30
_NPOS = 16
_BND = 32


def _dot_t(x, w):
    return jax.lax.dot_general(x, w, (((1,), (1,)), ((), ())),
                               preferred_element_type=jnp.float32)


def _dot_n(x, w):
    return jax.lax.dot_general(x, w, (((1,), (0,)), ((), ())),
                               preferred_element_type=jnp.float32)


def _ln(x, g, b):
    mu = jnp.mean(x, axis=1, keepdims=True)
    xc = x - mu
    var = jnp.mean(xc * xc, axis=1, keepdims=True)
    return xc * jax.lax.rsqrt(var + 1e-5) * g + b


def _heads_cat(fn, x):
    return jnp.concatenate([fn(x[:, h * _DH:(h + 1) * _DH]) for h in range(_NH)],
                           axis=1)


def _body(s_ref, m_ref, we_ref, be_ref, ws_ref, wout_ref, bout_ref,
          wq_ref, wke_ref, wks_ref, wkv_ref, wve_ref, wvs_ref, wvv_ref,
          wo_ref, w1_ref, b1_ref, w2_ref, b2_ref,
          ln1g_ref, ln1b_ref, ln2g_ref, ln2b_ref,
          fint_ref, fbnd_ref, mbnd_ref, bsel_ref, out_ref,
          rotk_ref, rotv_ref, hs_ref, hv_ref, mcat_ref, q_ref,
          kr_ref, vr_ref, ekb_ref, evb_ref, heb_ref, *, num_layers):
    N = s_ref.shape[1]
    NB = N
    num_batches = 1
    scale = 1.0 / math.sqrt(_DH)
    W = _BND

    s = s_ref[0]
    vocab = ws_ref.shape[0]
    onehot = (jax.lax.broadcasted_iota(jnp.int32, (NB, vocab), 1) == s
              ).astype(jnp.float32)
    hs_ref[...] = _dot_n(onehot, ws_ref[...])

    hEi = _dot_t(fint_ref[...], we_ref[...]) + be_ref[...]
    heb_ref[...] = _dot_t(fbnd_ref[...], we_ref[...]) + be_ref[...]

    seq_mask = m_ref[0]
    ones_int = jnp.ones((N - _BND, _K), jnp.float32)
    mcat_ref[...] = jnp.concatenate(
        sum([[mbnd_ref[...], ones_int] for _ in range(num_batches)], []), axis=0)

    hv_ref[...] = jnp.zeros((NB, _HID), jnp.float32)

    def layer_fn(l, carry):
        q_ref[...] = _dot_t(hv_ref[...], wq_ref[l]) * scale
        KP = _dot_t(hs_ref[...], wks_ref[l]) + _dot_t(hv_ref[...], wkv_ref[l])
        VP = _dot_t(hs_ref[...], wvs_ref[l]) + _dot_t(hv_ref[...], wvv_ref[l])
        EK = _dot_t(hEi, wke_ref[l])
        ekb_ref[...] = _dot_t(heb_ref[...], wke_ref[l])
        EV = _dot_t(hEi, wve_ref[l])
        evb_ref[...] = _dot_t(heb_ref[...], wve_ref[l])
        bnd = jnp.concatenate(
            [KP[b * N:b * N + _BND, :] for b in range(num_batches)], axis=0)
        bndv = jnp.concatenate(
            [VP[b * N:b * N + _BND, :] for b in range(num_batches)], axis=0)
        kr_ref[...] = _dot_n(bsel_ref[...], bnd)
        vr_ref[...] = _dot_n(bsel_ref[...], bndv)
        rotk_ref[0:NB] = KP
        rotv_ref[0:NB] = VP
        for r in range(1, 8):
            rotk_ref[r * NB:(r + 1) * NB] = jnp.concatenate(
                [KP[NB - r:, :], KP[:NB - r, :]], axis=0)
            rotv_ref[r * NB:(r + 1) * NB] = jnp.concatenate(
                [VP[NB - r:, :], VP[:NB - r, :]], axis=0)

        def band_row(k, q8, r8, rot_ref, r32_ref, e_row, eb_ref):
            ekb = eb_ref[pl.ds(k * _BND, _BND), :]
            p0 = r32_ref[pl.ds(k * W, _BND), :] + ekb
            p1 = rot_ref[pl.ds(r8 * NB + _BND - 8 * q8, NB - _BND), :] + e_row
            return jnp.concatenate([p0, p1], axis=0)

        def kbody(k, carry):
            m_run, s_run, acc = carry
            sh = k + 1
            q8 = sh // 8
            r8 = sh - 8 * q8
            ohr = (jax.lax.broadcasted_iota(jnp.int32, (1, _K), 1) == k
                   ).astype(jnp.float32)
            ohc = (jax.lax.broadcasted_iota(jnp.int32, (_K, 1), 0) == k
                   ).astype(jnp.float32)
            ek = _dot_n(ohr, EK)
            ev = _dot_n(ohr, EV)
            mcol = _dot_n(mcat_ref[...], ohc)
            kt = band_row(k, q8, r8, rotk_ref, kr_ref, ek, ekb_ref)
            prod = q_ref[...] * kt
            l4 = _heads_cat(
                lambda x: jnp.sum(x, axis=1, keepdims=True), prod)
            l4 = jnp.where(mcol > 0.0, l4, -1e9)
            m_new = jnp.maximum(m_run, l4)
            corr = jnp.exp(m_run - m_new)
            e = jnp.exp(l4 - m_new)
            s_new = s_run * corr + e
            w4 = e * mcol
            c128 = jnp.concatenate(
                [jnp.broadcast_to(corr[:, h:h + 1], (NB, _DH)) for h in range(_NH)],
                axis=1)
            a128 = jnp.concatenate(
                [jnp.broadcast_to(w4[:, h:h + 1], (NB, _DH)) for h in range(_NH)],
                axis=1)
            vt = band_row(k, q8, r8, rotv_ref, vr_ref, ev, evb_ref)
            return m_new, s_new, acc * c128 + a128 * vt

        m_run, s_run, acc = jax.lax.fori_loop(
            0, _K, kbody,
            (jnp.full((NB, _NH), -3.0e38, jnp.float32),
             jnp.zeros((NB, _NH), jnp.float32),
             jnp.zeros((NB, _HID), jnp.float32)))
        inv = 1.0 / s_run
        inv128 = jnp.concatenate(
            [jnp.broadcast_to(inv[:, h:h + 1], (NB, _DH)) for h in range(_NH)],
            axis=1)
        h_att = acc * inv128

        hV = _ln(hv_ref[...] + _dot_t(h_att, wo_ref[l]),
                 ln1g_ref[l], ln1b_ref[l])
        ffn = _dot_t(jax.nn.relu(_dot_t(hV, w1_ref[l]) + b1_ref[l]),
                     w2_ref[l]) + b2_ref[l]
        hV = _ln(hV + ffn, ln2g_ref[l], ln2b_ref[l])
        hv_ref[...] = hV * seq_mask
        return carry

    jax.lax.fori_loop(0, num_layers, layer_fn, 0)

    logits_out = _dot_t(hv_ref[...], wout_ref[...]) + bout_ref[...]
    mx = jnp.max(logits_out, axis=1, keepdims=True)
    lse = mx + jnp.log(jnp.sum(jnp.exp(logits_out - mx), axis=1, keepdims=True))
    out_ref[...] = (logits_out - lse).reshape(1, N, logits_out.shape[1])


def _band_constants(num_batches):
    freq = np.exp(np.arange(0, _NPOS, 2, dtype=np.float64)
                  * (-np.log(10000.0) / _NPOS))

    def feat(d):
        ang = d[..., None] * freq
        return np.concatenate([np.cos(ang), np.sin(ang)],
                              axis=-1).astype(np.float32)

    d_int = -(np.arange(_K, dtype=np.float64) + 1.0)
    feat_int = feat(d_int)

    nn = np.arange(_BND)[:, None]
    kk = np.arange(_K)[None, :]
    jsel = np.abs(nn - kk - 1)
    dbnd = (jsel - nn).astype(np.float64)
    feat_bnd = feat(dbnd.T).reshape(_K * _BND, _NPOS)
    mask_bnd = (dbnd < 0).astype(np.float32)

    bsel1 = np.zeros((_K, _BND, _BND), np.float32)
    for k in range(_K):
        for n in range(_BND):
            bsel1[k, n, jsel[n, k]] = 1.0
    W = _BND * num_batches
    bsel = np.zeros((_K, W, W), np.float32)
    for b in range(num_batches):
        bsel[:, b * _BND:(b + 1) * _BND, b * _BND:(b + 1) * _BND] = bsel1
    return feat_int, feat_bnd, mask_bnd, bsel.reshape(_K * W, W)


@jax.jit
def kernel(S, L, mask, params):
    del L
    p = params
    B, N = S.shape
    num_layers = p['W_Q'].shape[0]

    feat_int, feat_bnd, mask_bnd, bsel = _band_constants(1)

    wke = p['W_K'][:, :, 0:_HID]
    wks = p['W_K'][:, :, _HID:2 * _HID]
    wkv = p['W_K'][:, :, 2 * _HID:3 * _HID]
    wve = p['W_V'][:, :, 0:_HID]
    wvs = p['W_V'][:, :, _HID:2 * _HID]
    wvv = p['W_V'][:, :, 2 * _HID:3 * _HID]

    args = [
        S.reshape(B, N, 1), mask.reshape(B, N, 1),
        p['W_e'], p['b_e'].reshape(1, _HID),
        p['W_s'], p['W_out'], p['b_out'].reshape(1, -1),
        p['W_Q'], wke, wks, wkv, wve, wvs, wvv, p['W_O'],
        p['W_1'], p['b_1'].reshape(num_layers, 1, -1),
        p['W_2'], p['b_2'].reshape(num_layers, 1, -1),
        p['ln1_g'].reshape(num_layers, 1, -1),
        p['ln1_b'].reshape(num_layers, 1, -1),
        p['ln2_g'].reshape(num_layers, 1, -1),
        p['ln2_b'].reshape(num_layers, 1, -1),
        jnp.asarray(feat_int), jnp.asarray(feat_bnd),
        jnp.asarray(mask_bnd), jnp.asarray(bsel),
    ]
    full = [pl.BlockSpec(a.shape, functools.partial(lambda nd, b: (0,) * nd, a.ndim))
            for a in args[2:]]
    out = pl.pallas_call(
        functools.partial(_body, num_layers=num_layers),
        grid=(B,),
        in_specs=[pl.BlockSpec((1, N, 1), lambda b: (b, 0, 0)),
                  pl.BlockSpec((1, N, 1), lambda b: (b, 0, 0))] + full,
        out_specs=pl.BlockSpec((1, N, 20), lambda b: (b, 0, 0)),
        out_shape=jax.ShapeDtypeStruct((B, N, 20), jnp.float32),
        scratch_shapes=[pltpu.VMEM((8 * N, _HID), jnp.float32),
                        pltpu.VMEM((8 * N, _HID), jnp.float32),
                        pltpu.VMEM((N, _HID), jnp.float32),
                        pltpu.VMEM((N, _HID), jnp.float32),
                        pltpu.VMEM((N, _K), jnp.float32),
                        pltpu.VMEM((N, _HID), jnp.float32),
                        pltpu.VMEM((_K * _BND, _HID), jnp.float32),
                        pltpu.VMEM((_K * _BND, _HID), jnp.float32),
                        pltpu.VMEM((_K * _BND, _HID), jnp.float32),
                        pltpu.VMEM((_K * _BND, _HID), jnp.float32),
                        pltpu.VMEM((_K * _BND, _HID), jnp.float32)],
    )(*args)
    return out
